# baseline (speedup 1.0000x reference)
import functools

import jax
import jax.numpy as jnp
from jax import lax
from jax.experimental import pallas as pl
from jax.experimental.pallas import tpu as pltpu
from jax.experimental.pallas import tpu_sc as plsc

N = 10000
E = 320000
D = 128
NC = 2
NS = 16
NW = NC * NS
C = 128
CHN = 80
EW = C * CHN
E_PAD = NW * EW
RPT = 640
N_PAD = NS * RPT
RB = 1280


def _mesh():
    return plsc.VectorSubcoreMesh(core_axis_name="c", subcore_axis_name="s")


DW = 128


def _deg_body(src_hbm, ones_hbm, zeros_hbm, out_hbm, idx_v, ones_v, deg_sh,
              sem):
    c = lax.axis_index("c")
    s = lax.axis_index("s")
    wid = c * NS + s
    pltpu.sync_copy(zeros_hbm, deg_sh.at[pl.ds(s * RPT, RPT)])
    pltpu.sync_copy(src_hbm.at[wid], idx_v)
    pltpu.sync_copy(ones_hbm, ones_v)
    plsc.subcore_barrier()

    def body(j, _):
        pltpu.make_async_copy(ones_v, deg_sh.at[idx_v.at[j]],
                              sem).start(add=True)

        @pl.when(j >= 2)
        def _():
            pltpu.make_async_copy(ones_v, deg_sh.at[idx_v.at[j - 2]],
                                  sem).wait()

        return 0

    lax.fori_loop(0, CHN, body, 0)
    for t in (2, 1):
        pltpu.make_async_copy(ones_v, deg_sh.at[idx_v.at[CHN - t]],
                              sem).wait()
    plsc.subcore_barrier()
    pltpu.sync_copy(deg_sh.at[pl.ds(s * RPT, RPT)],
                    out_hbm.at[c, pl.ds(s * RPT, RPT)])


_deg_call = functools.partial(
    pl.kernel,
    out_type=jax.ShapeDtypeStruct((NC, N_PAD, DW), jnp.float32),
    mesh=_mesh(),
    scratch_types=[
        pltpu.VMEM((CHN, C), jnp.int32),
        pltpu.VMEM((C, DW), jnp.float32),
        pltpu.VMEM_SHARED((N_PAD, DW), jnp.float32),
        pltpu.SemaphoreType.DMA,
    ],
)(_deg_body)


CH = CHN // 2


def _scat_body(src_hbm, dst_hbm, u_hbm, out_hbm,
               src_v, dst_v, rows0_v, rows1_v, z_sh, sem0, sem1,
               ssem0, ssem1):
    c = lax.axis_index("c")
    s = lax.axis_index("s")
    wid = c * NS + s
    zeros16 = jnp.zeros((16,), jnp.float32)

    def zrow(i, _):
        for k in range(D // 16):
            rows0_v[i, pl.ds(k * 16, 16)] = zeros16
        return 0

    lax.fori_loop(0, C, zrow, 0)
    for r in range(RPT // C):
        pltpu.sync_copy(rows0_v, z_sh.at[pl.ds(s * RPT + r * C, C)])
    plsc.subcore_barrier()

    def gather(j, buf, sem):
        pltpu.async_copy(u_hbm.at[src_v.at[j]], buf, sem)

    def gwait(j, buf, sem):
        pltpu.make_async_copy(u_hbm.at[src_v.at[j]], buf, sem).wait()

    def scat(j, buf, sem):
        return pltpu.make_async_copy(buf, z_sh.at[dst_v.at[j]], sem)

    for h in (0, 1):
        pltpu.sync_copy(src_hbm.at[wid, pl.ds(h * CH, CH)], src_v)
        pltpu.sync_copy(dst_hbm.at[wid, pl.ds(h * CH, CH)], dst_v)
        gather(0, rows0_v, sem0)
        gather(1, rows1_v, sem1)

        def body(i, _):
            j0 = 2 * i
            j1 = 2 * i + 1
            gwait(j0, rows0_v, sem0)
            scat(j0, rows0_v, ssem0).start(add=True)
            gwait(j1, rows1_v, sem1)
            scat(j1, rows1_v, ssem1).start(add=True)

            @pl.when(j0 + 2 < CH)
            def _():
                scat(j0, rows0_v, ssem0).wait()
                gather(j0 + 2, rows0_v, sem0)

            @pl.when(j1 + 2 < CH)
            def _():
                scat(j1, rows1_v, ssem1).wait()
                gather(j1 + 2, rows1_v, sem1)

            return 0

        lax.fori_loop(0, CH // 2, body, 0)
        scat(CH - 2, rows0_v, ssem0).wait()
        scat(CH - 1, rows1_v, ssem1).wait()
    plsc.subcore_barrier()
    pltpu.sync_copy(z_sh.at[pl.ds(s * RPT, RPT)],
                    out_hbm.at[c, pl.ds(s * RPT, RPT)])


_scat_call = functools.partial(
    pl.kernel,
    out_type=jax.ShapeDtypeStruct((NC, N_PAD, D), jnp.float32),
    mesh=_mesh(),
    scratch_types=[
        pltpu.VMEM((CH, C), jnp.int32),
        pltpu.VMEM((CH, C), jnp.int32),
        pltpu.VMEM((C, D), jnp.float32),
        pltpu.VMEM((C, D), jnp.float32),
        pltpu.VMEM_SHARED((N_PAD, D), jnp.float32),
        pltpu.SemaphoreType.DMA,
        pltpu.SemaphoreType.DMA,
        pltpu.SemaphoreType.DMA,
        pltpu.SemaphoreType.DMA,
    ],
)(_scat_body)


def _dis_block(dega, degb):
    degs = dega[:, :1] + degb[:, :1]
    dis = jnp.where(degs > 0,
                    lax.rsqrt(jnp.maximum(degs, 1e-12)),
                    0.0)
    return degs, dis


def _tc1_body(xp_ref, dega_ref, degb_ref, w_ref, u0_ref, out0_ref):
    _, dis = _dis_block(dega_ref[...], degb_ref[...])
    xb = xp_ref[...]
    u0_ref[...] = dis * xb
    out0_ref[...] = jnp.dot(xb, w_ref[...], preferred_element_type=jnp.float32)


def _tc2_body(z0a_ref, z0b_ref, dega_ref, degb_ref, w_ref, out0_ref,
              u1_ref, acc_ref):
    zs = z0a_ref[...] + z0b_ref[...]
    degs, dis = _dis_block(dega_ref[...], degb_ref[...])
    u1_ref[...] = jnp.where(degs > 0, -(zs / jnp.maximum(degs, 1e-12)), 0.0)
    acc_ref[...] = out0_ref[...] - dis * jnp.dot(
        zs, w_ref[...], preferred_element_type=jnp.float32)


def _tc3_body(z1a_ref, z1b_ref, dega_ref, degb_ref, w_ref, acc_ref, b_ref,
              o_ref):
    zs = z1a_ref[...] + z1b_ref[...]
    _, dis = _dis_block(dega_ref[...], degb_ref[...])
    o_ref[...] = (acc_ref[...]
                  - 2.0 * dis * jnp.dot(zs, w_ref[...],
                                        preferred_element_type=jnp.float32)
                  + b_ref[...])


def _row_spec():
    return pl.BlockSpec((RB, D), lambda i: (i, 0))


def _degt_spec():
    return pl.BlockSpec((RB, D), lambda i: (i, 0))


def _w_spec():
    return pl.BlockSpec((D, D), lambda i: (0, 0))


_GRID = (N_PAD // RB,)


def kernel(x, edge_index, Ws, bias):
    src = edge_index[0]
    dst = edge_index[1]
    sl = src == dst
    srcm = jnp.where(sl, N, src).astype(jnp.int32)
    dstm = jnp.where(sl, N, dst).astype(jnp.int32)
    ewr = E // NW
    padw = EW - ewr
    pad_idx = jnp.broadcast_to((N + jnp.arange(padw, dtype=jnp.int32))[None],
                               (NW, padw))
    srcm_w = jnp.concatenate(
        [srcm.reshape(NW, ewr), pad_idx], axis=1).reshape(NW, CHN, C)
    src_w = jnp.concatenate(
        [src.astype(jnp.int32).reshape(NW, ewr), pad_idx],
        axis=1).reshape(NW, CHN, C)
    dst_w = jnp.concatenate(
        [dstm.reshape(NW, ewr), pad_idx], axis=1).reshape(NW, CHN, C)
    x_pad = jnp.zeros((N_PAD, D), jnp.float32).at[:N].set(x)
    zeros_blk = jnp.zeros((RPT, D), jnp.float32)
    ones_blk = jnp.ones((C, DW), jnp.float32)
    w02 = Ws[0] - Ws[2]

    degp = _deg_call(srcm_w, ones_blk, zeros_blk)
    dega, degb = degp[0], degp[1]

    u0, out0 = pl.pallas_call(
        _tc1_body,
        grid=_GRID,
        in_specs=[_row_spec(), _degt_spec(), _degt_spec(), _w_spec()],
        out_specs=[_row_spec(), _row_spec()],
        out_shape=[jax.ShapeDtypeStruct((N_PAD, D), jnp.float32),
                   jax.ShapeDtypeStruct((N_PAD, D), jnp.float32)],
    )(x_pad, dega, degb, w02)

    z0p = _scat_call(src_w, dst_w, u0)

    u1, acc1 = pl.pallas_call(
        _tc2_body,
        grid=_GRID,
        in_specs=[_row_spec(), _row_spec(), _degt_spec(), _degt_spec(),
                  _w_spec(), _row_spec()],
        out_specs=[_row_spec(), _row_spec()],
        out_shape=[jax.ShapeDtypeStruct((N_PAD, D), jnp.float32),
                   jax.ShapeDtypeStruct((N_PAD, D), jnp.float32)],
    )(z0p[0], z0p[1], dega, degb, Ws[1], out0)

    z1p = _scat_call(src_w, dst_w, u1)

    out = pl.pallas_call(
        _tc3_body,
        grid=_GRID,
        in_specs=[_row_spec(), _row_spec(), _degt_spec(), _degt_spec(),
                  _w_spec(), _row_spec(), pl.BlockSpec((1, D), lambda i: (0, 0))],
        out_specs=_row_spec(),
        out_shape=jax.ShapeDtypeStruct((N_PAD, D), jnp.float32),
    )(z1p[0], z1p[1], dega, degb, Ws[2], acc1, bias[None, :])

    return out[:N]


# speedup vs baseline: 1.1861x; 1.1861x over previous
import functools

import jax
import jax.numpy as jnp
from jax import lax
from jax.experimental import pallas as pl
from jax.experimental.pallas import tpu as pltpu
from jax.experimental.pallas import tpu_sc as plsc

N = 10000
E = 320000
D = 128
NC = 2
NS = 16
NW = NC * NS
C = 128
CHN = 80
EW = C * CHN
E_PAD = NW * EW
RPT = 640
N_PAD = NS * RPT
RB = 1280


def _mesh():
    return plsc.VectorSubcoreMesh(core_axis_name="c", subcore_axis_name="s")


DW = 128


def _deg_body(src_hbm, ones_hbm, zeros_hbm, out_hbm, idx_v, ones_v, deg_sh,
              sem):
    c = lax.axis_index("c")
    s = lax.axis_index("s")
    wid = c * NS + s
    pltpu.sync_copy(zeros_hbm, deg_sh.at[pl.ds(s * RPT, RPT)])
    pltpu.sync_copy(src_hbm.at[wid], idx_v)
    pltpu.sync_copy(ones_hbm, ones_v)
    plsc.subcore_barrier()

    def body(j, _):
        pltpu.sync_copy(ones_v, deg_sh.at[idx_v.at[j]], add=True)
        return 0

    lax.fori_loop(0, CHN, body, 0)
    plsc.subcore_barrier()
    pltpu.sync_copy(deg_sh.at[pl.ds(s * RPT, RPT)],
                    out_hbm.at[c, pl.ds(s * RPT, RPT)])


_deg_call = functools.partial(
    pl.kernel,
    out_type=jax.ShapeDtypeStruct((NC, N_PAD, DW), jnp.float32),
    mesh=_mesh(),
    scratch_types=[
        pltpu.VMEM((CHN, C), jnp.int32),
        pltpu.VMEM((C, DW), jnp.float32),
        pltpu.VMEM_SHARED((N_PAD, DW), jnp.float32),
        pltpu.SemaphoreType.DMA,
    ],
)(_deg_body)


CH = CHN // 2


def _scat_body(src_hbm, dst_hbm, u_hbm, out_hbm,
               src_v, dst_v, rows0_v, rows1_v, z_sh, sem0, sem1):
    c = lax.axis_index("c")
    s = lax.axis_index("s")
    wid = c * NS + s
    zeros16 = jnp.zeros((16,), jnp.float32)

    def zrow(i, _):
        for k in range(D // 16):
            rows0_v[i, pl.ds(k * 16, 16)] = zeros16
        return 0

    lax.fori_loop(0, C, zrow, 0)
    for r in range(RPT // C):
        pltpu.sync_copy(rows0_v, z_sh.at[pl.ds(s * RPT + r * C, C)])
    plsc.subcore_barrier()

    def gather(j, buf, sem):
        pltpu.async_copy(u_hbm.at[src_v.at[j]], buf, sem)

    def gwait(j, buf, sem):
        pltpu.make_async_copy(u_hbm.at[src_v.at[j]], buf, sem).wait()

    for h in (0, 1):
        pltpu.sync_copy(src_hbm.at[wid, pl.ds(h * CH, CH)], src_v)
        pltpu.sync_copy(dst_hbm.at[wid, pl.ds(h * CH, CH)], dst_v)
        gather(0, rows0_v, sem0)
        gather(1, rows1_v, sem1)

        def body(i, _):
            j0 = 2 * i
            j1 = 2 * i + 1
            gwait(j0, rows0_v, sem0)
            pltpu.sync_copy(rows0_v, z_sh.at[dst_v.at[j0]], add=True)

            @pl.when(j0 + 2 < CH)
            def _():
                gather(j0 + 2, rows0_v, sem0)

            gwait(j1, rows1_v, sem1)
            pltpu.sync_copy(rows1_v, z_sh.at[dst_v.at[j1]], add=True)

            @pl.when(j1 + 2 < CH)
            def _():
                gather(j1 + 2, rows1_v, sem1)

            return 0

        lax.fori_loop(0, CH // 2, body, 0)
    plsc.subcore_barrier()
    pltpu.sync_copy(z_sh.at[pl.ds(s * RPT, RPT)],
                    out_hbm.at[c, pl.ds(s * RPT, RPT)])


_scat_call = functools.partial(
    pl.kernel,
    out_type=jax.ShapeDtypeStruct((NC, N_PAD, D), jnp.float32),
    mesh=_mesh(),
    scratch_types=[
        pltpu.VMEM((CH, C), jnp.int32),
        pltpu.VMEM((CH, C), jnp.int32),
        pltpu.VMEM((C, D), jnp.float32),
        pltpu.VMEM((C, D), jnp.float32),
        pltpu.VMEM_SHARED((N_PAD, D), jnp.float32),
        pltpu.SemaphoreType.DMA,
        pltpu.SemaphoreType.DMA,
    ],
)(_scat_body)


def _dis_block(dega, degb):
    degs = dega[:, :1] + degb[:, :1]
    dis = jnp.where(degs > 0,
                    lax.rsqrt(jnp.maximum(degs, 1e-12)),
                    0.0)
    return degs, dis


def _tc1_body(xp_ref, dega_ref, degb_ref, w_ref, u0_ref, out0_ref):
    _, dis = _dis_block(dega_ref[...], degb_ref[...])
    xb = xp_ref[...]
    u0_ref[...] = dis * xb
    out0_ref[...] = jnp.dot(xb, w_ref[...], preferred_element_type=jnp.float32)


def _tc2_body(z0a_ref, z0b_ref, dega_ref, degb_ref, w_ref, out0_ref,
              u1_ref, acc_ref):
    zs = z0a_ref[...] + z0b_ref[...]
    degs, dis = _dis_block(dega_ref[...], degb_ref[...])
    u1_ref[...] = jnp.where(degs > 0, -(zs / jnp.maximum(degs, 1e-12)), 0.0)
    acc_ref[...] = out0_ref[...] - dis * jnp.dot(
        zs, w_ref[...], preferred_element_type=jnp.float32)


def _tc3_body(z1a_ref, z1b_ref, dega_ref, degb_ref, w_ref, acc_ref, b_ref,
              o_ref):
    zs = z1a_ref[...] + z1b_ref[...]
    _, dis = _dis_block(dega_ref[...], degb_ref[...])
    o_ref[...] = (acc_ref[...]
                  - 2.0 * dis * jnp.dot(zs, w_ref[...],
                                        preferred_element_type=jnp.float32)
                  + b_ref[...])


def _row_spec():
    return pl.BlockSpec((RB, D), lambda i: (i, 0))


def _degt_spec():
    return pl.BlockSpec((RB, DW), lambda i: (i, 0))


def _w_spec():
    return pl.BlockSpec((D, D), lambda i: (0, 0))


_GRID = (N_PAD // RB,)


def kernel(x, edge_index, Ws, bias):
    src = edge_index[0]
    dst = edge_index[1]
    sl = src == dst
    srcm = jnp.where(sl, N, src).astype(jnp.int32)
    dstm = jnp.where(sl, N, dst).astype(jnp.int32)
    ewr = E // NW
    padw = EW - ewr
    pad_idx = jnp.broadcast_to((N + jnp.arange(padw, dtype=jnp.int32))[None],
                               (NW, padw))
    srcm_w = jnp.concatenate(
        [srcm.reshape(NW, ewr), pad_idx], axis=1).reshape(NW, CHN, C)
    src_w = jnp.concatenate(
        [src.astype(jnp.int32).reshape(NW, ewr), pad_idx],
        axis=1).reshape(NW, CHN, C)
    dst_w = jnp.concatenate(
        [dstm.reshape(NW, ewr), pad_idx], axis=1).reshape(NW, CHN, C)
    x_pad = jnp.zeros((N_PAD, D), jnp.float32).at[:N].set(x)
    zeros_blk = jnp.zeros((RPT, D), jnp.float32)
    ones_blk = jnp.ones((C, DW), jnp.float32)
    w02 = Ws[0] - Ws[2]

    degp = _deg_call(srcm_w, ones_blk, zeros_blk)
    dega, degb = degp[0], degp[1]

    u0, out0 = pl.pallas_call(
        _tc1_body,
        grid=_GRID,
        in_specs=[_row_spec(), _degt_spec(), _degt_spec(), _w_spec()],
        out_specs=[_row_spec(), _row_spec()],
        out_shape=[jax.ShapeDtypeStruct((N_PAD, D), jnp.float32),
                   jax.ShapeDtypeStruct((N_PAD, D), jnp.float32)],
    )(x_pad, dega, degb, w02)

    z0p = _scat_call(src_w, dst_w, u0)

    u1, acc1 = pl.pallas_call(
        _tc2_body,
        grid=_GRID,
        in_specs=[_row_spec(), _row_spec(), _degt_spec(), _degt_spec(),
                  _w_spec(), _row_spec()],
        out_specs=[_row_spec(), _row_spec()],
        out_shape=[jax.ShapeDtypeStruct((N_PAD, D), jnp.float32),
                   jax.ShapeDtypeStruct((N_PAD, D), jnp.float32)],
    )(z0p[0], z0p[1], dega, degb, Ws[1], out0)

    z1p = _scat_call(src_w, dst_w, u1)

    out = pl.pallas_call(
        _tc3_body,
        grid=_GRID,
        in_specs=[_row_spec(), _row_spec(), _degt_spec(), _degt_spec(),
                  _w_spec(), _row_spec(), pl.BlockSpec((1, D), lambda i: (0, 0))],
        out_specs=_row_spec(),
        out_shape=jax.ShapeDtypeStruct((N_PAD, D), jnp.float32),
    )(z1p[0], z1p[1], dega, degb, Ws[2], acc1, bias[None, :])

    return out[:N]
